# baseline (speedup 1.0000x reference)
import jax
import jax.numpy as jnp
from jax import lax
from jax.experimental import pallas as pl
from jax.experimental.pallas import tpu as pltpu
from jax.experimental.pallas import tpu_sc as plsc

B, S, H, V, P, T = 4, 512, 768, 30522, 512, 2
N = B * S
NW = 32
TPW = N // NW
PPW = S // NW
LANES = 16
NCH = H // LANES
UNROLL = 4


def _rsqrt(x):
    xb = lax.bitcast_convert_type(x, jnp.int32)
    yb = jnp.int32(0x5F3759DF) - lax.shift_right_logical(xb, 1)
    y = lax.bitcast_convert_type(yb, jnp.float32)
    for _ in range(3):
        y = y * (1.5 - 0.5 * x * y * y)
    return y


def _sc_body(idtt_hbm, word_hbm, aux_hbm,
             out_hbm, idx_v, tt_v, rows_v, pos01_v, tte_v, gamma_v, beta_v,
             sem, osem):
    c = lax.axis_index("c")
    s = lax.axis_index("s")
    wid = s * 2 + c
    pbase = wid * PPW

    ics = [pltpu.async_copy(idtt_hbm.at[0, pl.ds(b * S + pbase, PPW)],
                            idx_v.at[pl.ds(b * PPW, PPW)], osem)
           for b in range(B)]
    for cp in ics:
        cp.wait()
    gathers = [
        pltpu.async_copy(word_hbm.at[idx_v.at[pl.ds(h * (TPW // 2), TPW // 2)]],
                         rows_v.at[pl.ds(h * (TPW // 2), TPW // 2)], sem)
        for h in range(2)
    ]
    for b in range(B):
        pltpu.sync_copy(idtt_hbm.at[1, pl.ds(b * S + pbase, PPW)],
                        tt_v.at[pl.ds(b * PPW, PPW)])
    pltpu.sync_copy(aux_hbm.at[pl.ds(pbase, PPW)], pos01_v.at[0])
    pltpu.sync_copy(aux_hbm.at[pl.ds(pbase, PPW)], pos01_v.at[1])
    pltpu.sync_copy(aux_hbm.at[pl.ds(P, T)], tte_v)
    pltpu.sync_copy(aux_hbm.at[P + T], gamma_v)
    pltpu.sync_copy(aux_hbm.at[P + T + 1], beta_v)

    def precomb(k, _):
        @plsc.parallel_loop(0, NCH, unroll=8)
        def _(j):
            sl = pl.ds(j * LANES, LANES)
            pos01_v[0, k, sl] = pos01_v[0, k, sl] + tte_v[0, sl]
            pos01_v[1, k, sl] = pos01_v[1, k, sl] + tte_v[1, sl]
        return 0
    lax.fori_loop(0, PPW, precomb, 0)

    lane = jnp.arange(LANES, dtype=jnp.int32)
    zero = jnp.zeros((LANES,), jnp.float32)

    outs = []
    for b in range(B):
        if b % 2 == 0:
            gathers[b // 2].wait()
        tt16 = tt_v[pl.ds(b * PPW, LANES)]

        def token_body(kk, _, b=b, tt16=tt16):
            i = b * PPW + kk
            tsel = jnp.sum(jnp.where(lane == kk, tt16, 0))

            def p1(j, car):
                acc, acc2 = car
                sl = pl.ds(j * LANES, LANES)
                e = rows_v[i, sl] + pos01_v[tsel, kk, sl]
                rows_v[i, sl] = e
                return (acc + e, acc2 + e * e)
            acc, acc2 = plsc.parallel_loop(0, NCH, unroll=8,
                                           carry=(zero, zero))(p1)
            mean = jnp.sum(acc) * (1.0 / H)
            var = jnp.sum(acc2) * (1.0 / H) - mean * mean
            rstd = _rsqrt(var + 1e-12)
            nmean = mean * rstd

            @plsc.parallel_loop(0, NCH, unroll=8)
            def _(j):
                sl = pl.ds(j * LANES, LANES)
                rows_v[i, sl] = (rows_v[i, sl] * rstd - nmean) * gamma_v[sl] \
                    + beta_v[sl]
            return 0
        lax.fori_loop(0, PPW, token_body, 0)
        outs.append(pltpu.async_copy(rows_v.at[pl.ds(b * PPW, PPW)],
                                     out_hbm.at[pl.ds(b * S + pbase, PPW)],
                                     osem))
    for cp in outs:
        cp.wait()


@jax.jit
def kernel(input_ids, token_type_ids, word_embeddings, position_embeddings,
           token_type_embeddings, ln_gamma, ln_beta):
    mesh = plsc.VectorSubcoreMesh(core_axis_name="c", subcore_axis_name="s")
    k = pl.kernel(
        _sc_body,
        out_type=jax.ShapeDtypeStruct((N, H), jnp.float32),
        mesh=mesh,
        compiler_params=pltpu.CompilerParams(needs_layout_passes=False),
        scratch_types=[
            pltpu.VMEM((TPW,), jnp.int32),
            pltpu.VMEM((TPW + LANES,), jnp.int32),
            pltpu.VMEM((TPW, H), jnp.float32),
            pltpu.VMEM((T, PPW, H), jnp.float32),
            pltpu.VMEM((T, H), jnp.float32),
            pltpu.VMEM((H,), jnp.float32),
            pltpu.VMEM((H,), jnp.float32),
            pltpu.SemaphoreType.DMA,
            pltpu.SemaphoreType.DMA,
        ],
    )
    idtt = jnp.stack([input_ids.reshape(N), token_type_ids.reshape(N)])
    aux = jnp.concatenate([position_embeddings, token_type_embeddings,
                           ln_gamma[None], ln_beta[None]])
    out = k(idtt, word_embeddings, aux)
    return out.reshape(B, S, H)


# speedup vs baseline: 1.0713x; 1.0713x over previous
import jax
import jax.numpy as jnp
from jax import lax
from jax.experimental import pallas as pl
from jax.experimental.pallas import tpu as pltpu
from jax.experimental.pallas import tpu_sc as plsc

B, S, H, V, P, T = 4, 512, 768, 30522, 512, 2
N = B * S
NW = 32
TPW = N // NW
PPW = S // NW
LANES = 16
NCH = H // LANES
UNROLL = 4


def _rsqrt(x):
    xb = lax.bitcast_convert_type(x, jnp.int32)
    yb = jnp.int32(0x5F3759DF) - lax.shift_right_logical(xb, 1)
    y = lax.bitcast_convert_type(yb, jnp.float32)
    for _ in range(3):
        y = y * (1.5 - 0.5 * x * y * y)
    return y


def _sc_body(ids_hbm, tt_hbm, word_hbm, pos_hbm, tte_hbm, gamma_hbm, beta_hbm,
             out_hbm, idx_v, tt_v, rows_v, pos01_v, tte_v, gamma_v, beta_v,
             sem, osem):
    c = lax.axis_index("c")
    s = lax.axis_index("s")
    wid = s * 2 + c
    pbase = wid * PPW

    ics = [pltpu.async_copy(ids_hbm.at[pl.ds(b * S + pbase, PPW)],
                            idx_v.at[pl.ds(b * PPW, PPW)], osem)
           for b in range(B)]
    gathers = []
    for b in range(B):
        ics[b].wait()
        gathers.append(
            pltpu.async_copy(word_hbm.at[idx_v.at[pl.ds(b * PPW, PPW)]],
                             rows_v.at[pl.ds(b * PPW, PPW)], sem))
    for b in range(B):
        pltpu.sync_copy(tt_hbm.at[pl.ds(b * S + pbase, PPW)],
                        tt_v.at[pl.ds(b * PPW, PPW)])
    pltpu.sync_copy(pos_hbm.at[pl.ds(pbase, PPW)], pos01_v.at[0])
    pltpu.sync_copy(pos_hbm.at[pl.ds(pbase, PPW)], pos01_v.at[1])
    pltpu.sync_copy(tte_hbm, tte_v)
    pltpu.sync_copy(gamma_hbm, gamma_v)
    pltpu.sync_copy(beta_hbm, beta_v)

    def precomb(k, _):
        @plsc.parallel_loop(0, NCH, unroll=8)
        def _(j):
            sl = pl.ds(j * LANES, LANES)
            pos01_v[0, k, sl] = pos01_v[0, k, sl] + tte_v[0, sl]
            pos01_v[1, k, sl] = pos01_v[1, k, sl] + tte_v[1, sl]
        return 0
    lax.fori_loop(0, PPW, precomb, 0)

    lane = jnp.arange(LANES, dtype=jnp.int32)
    zero = jnp.zeros((LANES,), jnp.float32)

    outs = []
    for b in range(B):
        gathers[b].wait()
        tt16 = tt_v[pl.ds(b * PPW, LANES)]

        def token_body(kk, _, b=b, tt16=tt16):
            i = b * PPW + kk
            tsel = jnp.sum(jnp.where(lane == kk, tt16, 0))

            def p1(j, car):
                acc, acc2 = car
                sl = pl.ds(j * LANES, LANES)
                e = rows_v[i, sl] + pos01_v[tsel, kk, sl]
                rows_v[i, sl] = e
                return (acc + e, acc2 + e * e)
            acc, acc2 = plsc.parallel_loop(0, NCH, unroll=8,
                                           carry=(zero, zero))(p1)
            mean = jnp.sum(acc) * (1.0 / H)
            var = jnp.sum(acc2) * (1.0 / H) - mean * mean
            rstd = _rsqrt(var + 1e-12)
            nmean = mean * rstd

            @plsc.parallel_loop(0, NCH, unroll=8)
            def _(j):
                sl = pl.ds(j * LANES, LANES)
                rows_v[i, sl] = (rows_v[i, sl] * rstd - nmean) * gamma_v[sl] \
                    + beta_v[sl]
            return 0
        lax.fori_loop(0, PPW, token_body, 0)
        outs.append(pltpu.async_copy(rows_v.at[pl.ds(b * PPW, PPW)],
                                     out_hbm.at[pl.ds(b * S + pbase, PPW)],
                                     osem))
    for cp in outs:
        cp.wait()


@jax.jit
def kernel(input_ids, token_type_ids, word_embeddings, position_embeddings,
           token_type_embeddings, ln_gamma, ln_beta):
    mesh = plsc.VectorSubcoreMesh(core_axis_name="c", subcore_axis_name="s")
    k = pl.kernel(
        _sc_body,
        out_type=jax.ShapeDtypeStruct((N, H), jnp.float32),
        mesh=mesh,
        compiler_params=pltpu.CompilerParams(needs_layout_passes=False),
        scratch_types=[
            pltpu.VMEM((TPW,), jnp.int32),
            pltpu.VMEM((TPW + LANES,), jnp.int32),
            pltpu.VMEM((TPW, H), jnp.float32),
            pltpu.VMEM((T, PPW, H), jnp.float32),
            pltpu.VMEM((T, H), jnp.float32),
            pltpu.VMEM((H,), jnp.float32),
            pltpu.VMEM((H,), jnp.float32),
            pltpu.SemaphoreType.DMA,
            pltpu.SemaphoreType.DMA,
        ],
    )
    out = k(input_ids.reshape(N), token_type_ids.reshape(N),
            word_embeddings, position_embeddings, token_type_embeddings,
            ln_gamma, ln_beta)
    return out.reshape(B, S, H)


# speedup vs baseline: 1.1481x; 1.0717x over previous
import jax
import jax.numpy as jnp
from jax import lax
from jax.experimental import pallas as pl
from jax.experimental.pallas import tpu as pltpu
from jax.experimental.pallas import tpu_sc as plsc

B, S, H, V, P, T = 4, 512, 768, 30522, 512, 2
N = B * S
NW = 32
TPW = N // NW
PPW = S // NW
LANES = 16
NCH = H // LANES
UNROLL = 4


def _rsqrt(x):
    xb = lax.bitcast_convert_type(x, jnp.int32)
    yb = jnp.int32(0x5F3759DF) - lax.shift_right_logical(xb, 1)
    y = lax.bitcast_convert_type(yb, jnp.float32)
    for _ in range(3):
        y = y * (1.5 - 0.5 * x * y * y)
    return y


def _sc_body(ids_hbm, tt_hbm, word_hbm, pos_hbm, tte_hbm, gamma_hbm, beta_hbm,
             out_hbm, idx_v, tt_v, rows_v, pos01_v, tte_v, gamma_v, beta_v,
             sem, osem):
    c = lax.axis_index("c")
    s = lax.axis_index("s")
    wid = s * 2 + c
    pbase = wid * PPW

    ics = [pltpu.async_copy(ids_hbm.at[pl.ds(b * S + pbase, PPW)],
                            idx_v.at[pl.ds(b * PPW, PPW)], osem)
           for b in range(B)]
    gathers = []
    for b in range(B):
        ics[b].wait()
        gathers.append(
            pltpu.async_copy(word_hbm.at[idx_v.at[pl.ds(b * PPW, PPW)]],
                             rows_v.at[pl.ds(b * PPW, PPW)], sem))
    for b in range(B):
        pltpu.sync_copy(tt_hbm.at[pl.ds(b * S + pbase, PPW)],
                        tt_v.at[pl.ds(b * PPW, PPW)])
    pltpu.sync_copy(pos_hbm.at[pl.ds(pbase, PPW)], pos01_v.at[0])
    pltpu.sync_copy(pos_hbm.at[pl.ds(pbase, PPW)], pos01_v.at[1])
    pltpu.sync_copy(tte_hbm, tte_v)
    pltpu.sync_copy(gamma_hbm, gamma_v)
    pltpu.sync_copy(beta_hbm, beta_v)

    def precomb(k, _):
        @plsc.parallel_loop(0, NCH, unroll=8)
        def _(j):
            sl = pl.ds(j * LANES, LANES)
            pos01_v[0, k, sl] = pos01_v[0, k, sl] + tte_v[0, sl]
            pos01_v[1, k, sl] = pos01_v[1, k, sl] + tte_v[1, sl]
        return 0
    lax.fori_loop(0, PPW, precomb, 0)

    lane = jnp.arange(LANES, dtype=jnp.int32)
    zero = jnp.zeros((LANES,), jnp.float32)

    outs = []
    for b in range(B):
        gathers[b].wait()
        tt16 = tt_v[pl.ds(b * PPW, LANES)]

        def token_body(kk, _, b=b, tt16=tt16):
            ka = 2 * kk
            kb = ka + 1
            ia = b * PPW + ka
            ib = ia + 1
            tsa = jnp.sum(jnp.where(lane == ka, tt16, 0))
            tsb = jnp.sum(jnp.where(lane == kb, tt16, 0))

            def p1(j, car):
                aa, aa2, ab, ab2 = car
                sl = pl.ds(j * LANES, LANES)
                ea = rows_v[ia, sl] + pos01_v[tsa, ka, sl]
                eb = rows_v[ib, sl] + pos01_v[tsb, kb, sl]
                rows_v[ia, sl] = ea
                rows_v[ib, sl] = eb
                return (aa + ea, aa2 + ea * ea, ab + eb, ab2 + eb * eb)
            aa, aa2, ab, ab2 = plsc.parallel_loop(
                0, NCH, unroll=4, carry=(zero, zero, zero, zero))(p1)
            mean_a = jnp.sum(aa) * (1.0 / H)
            mean_b = jnp.sum(ab) * (1.0 / H)
            var_a = jnp.sum(aa2) * (1.0 / H) - mean_a * mean_a
            var_b = jnp.sum(ab2) * (1.0 / H) - mean_b * mean_b
            rstd_a = _rsqrt(var_a + 1e-12)
            rstd_b = _rsqrt(var_b + 1e-12)
            nmean_a = mean_a * rstd_a
            nmean_b = mean_b * rstd_b

            @plsc.parallel_loop(0, NCH, unroll=4)
            def _(j):
                sl = pl.ds(j * LANES, LANES)
                g = gamma_v[sl]
                bt = beta_v[sl]
                rows_v[ia, sl] = (rows_v[ia, sl] * rstd_a - nmean_a) * g + bt
                rows_v[ib, sl] = (rows_v[ib, sl] * rstd_b - nmean_b) * g + bt
            return 0
        lax.fori_loop(0, PPW // 2, token_body, 0)
        outs.append(pltpu.async_copy(rows_v.at[pl.ds(b * PPW, PPW)],
                                     out_hbm.at[pl.ds(b * S + pbase, PPW)],
                                     osem))
    for cp in outs:
        cp.wait()


@jax.jit
def kernel(input_ids, token_type_ids, word_embeddings, position_embeddings,
           token_type_embeddings, ln_gamma, ln_beta):
    mesh = plsc.VectorSubcoreMesh(core_axis_name="c", subcore_axis_name="s")
    k = pl.kernel(
        _sc_body,
        out_type=jax.ShapeDtypeStruct((N, H), jnp.float32),
        mesh=mesh,
        compiler_params=pltpu.CompilerParams(needs_layout_passes=False),
        scratch_types=[
            pltpu.VMEM((TPW,), jnp.int32),
            pltpu.VMEM((TPW + LANES,), jnp.int32),
            pltpu.VMEM((TPW, H), jnp.float32),
            pltpu.VMEM((T, PPW, H), jnp.float32),
            pltpu.VMEM((T, H), jnp.float32),
            pltpu.VMEM((H,), jnp.float32),
            pltpu.VMEM((H,), jnp.float32),
            pltpu.SemaphoreType.DMA,
            pltpu.SemaphoreType.DMA,
        ],
    )
    out = k(input_ids.reshape(N), token_type_ids.reshape(N),
            word_embeddings, position_embeddings, token_type_embeddings,
            ln_gamma, ln_beta)
    return out.reshape(B, S, H)
